# Initial kernel scaffold; baseline (speedup 1.0000x reference)
import functools

import jax
import jax.numpy as jnp
from jax import lax
from jax.experimental import pallas as pl
from jax.experimental.pallas import tpu as pltpu
from jax.experimental.pallas import tpu_sc as plsc

N = 50000
E = 1600000
G = 128
POS = 3
F = 5
XD = POS + F
MD = 16
PD = 20
NC = 2
NS = 16
NW = NC * NS
C = 100
K = 10
SCH = C * K
EPW = E // NW
NSUP = EPW // SCH
RPW = EPW // C
NPT = N // NS

_MESH = plsc.VectorSubcoreMesh(core_axis_name="c", subcore_axis_name="s")


def _gather_body(x_hbm, si_hbm, di_hbm, gs_hbm, gd_hbm,
                 idx_s, idx_d, rows_s, rows_d, sem_s, sem_d):
    c = lax.axis_index("c")
    s = lax.axis_index("s")
    wid = s * NC + c

    def step(j, carry):
        r0 = wid * RPW + j * K
        e0 = r0 * C
        pltpu.sync_copy(si_hbm.at[pl.ds(r0, K)], idx_s)
        pltpu.sync_copy(di_hbm.at[pl.ds(r0, K)], idx_d)
        cps = [pltpu.async_copy(x_hbm.at[idx_s.at[k]],
                                rows_s.at[pl.ds(k * C, C)], sem_s)
               for k in range(K)]
        cpd = [pltpu.async_copy(x_hbm.at[idx_d.at[k]],
                                rows_d.at[pl.ds(k * C, C)], sem_d)
               for k in range(K)]
        for cp in cps:
            cp.wait()
        for cp in cpd:
            cp.wait()
        pltpu.sync_copy(rows_s, gs_hbm.at[pl.ds(e0, SCH)])
        pltpu.sync_copy(rows_d, gd_hbm.at[pl.ds(e0, SCH)])
        return carry

    lax.fori_loop(0, NSUP, step, 0)


_gather = pl.kernel(
    _gather_body,
    out_type=(jax.ShapeDtypeStruct((E, XD), jnp.float32),
              jax.ShapeDtypeStruct((E, XD), jnp.float32)),
    mesh=_MESH,
    scratch_types=[
        pltpu.VMEM((K, C), jnp.int32),
        pltpu.VMEM((K, C), jnp.int32),
        pltpu.VMEM((SCH, XD), jnp.float32),
        pltpu.VMEM((SCH, XD), jnp.float32),
        pltpu.SemaphoreType.DMA,
        pltpu.SemaphoreType.DMA,
    ],
)


def _scatter_body(p_hbm, di_hbm, z_hbm, out_hbm, idx_d, prow, acc, sem):
    c = lax.axis_index("c")
    s = lax.axis_index("s")
    wid = s * NC + c

    pltpu.sync_copy(z_hbm, acc.at[pl.ds(s * NPT, NPT)])
    plsc.subcore_barrier()

    def step(j, carry):
        r0 = wid * RPW + j * K
        e0 = r0 * C
        pltpu.sync_copy(di_hbm.at[pl.ds(r0, K)], idx_d)
        pltpu.sync_copy(p_hbm.at[pl.ds(e0, SCH)], prow)
        cps = [pltpu.async_copy(prow.at[pl.ds(k * C, C)],
                                acc.at[idx_d.at[k]], sem, add=True)
               for k in range(K)]
        for cp in cps:
            cp.wait()
        return carry

    lax.fori_loop(0, NSUP, step, 0)
    plsc.subcore_barrier()
    pltpu.sync_copy(acc.at[pl.ds(s * NPT, NPT)],
                    out_hbm.at[c].at[pl.ds(s * NPT, NPT)])


_scatter = pl.kernel(
    _scatter_body,
    out_type=jax.ShapeDtypeStruct((NC, N, PD), jnp.float32),
    mesh=_MESH,
    scratch_types=[
        pltpu.VMEM((K, C), jnp.int32),
        pltpu.VMEM((SCH, PD), jnp.float32),
        pltpu.VMEM_SHARED((N, PD), jnp.float32),
        pltpu.SemaphoreType.DMA,
    ],
)


BE = 2000


def _edge_body(gs, gd, ea, w1, b1, w2, b2, cw1, cb1, cw2, cb2, sw, sb,
               cscale, p_out):
    def mm_r(w, x):
        return lax.dot_general(w, x, (((1,), (1,)), ((), ())),
                               preferred_element_type=jnp.float32)

    def mm_f(w, x):
        return lax.dot_general(w, x, (((1,), (0,)), ((), ())),
                               preferred_element_type=jnp.float32)

    xs = gs[...]
    xd = gd[...]
    rel = xs[:, :POS] - xd[:, :POS]
    rel_dist = jnp.sum(rel * rel, axis=1, keepdims=True)
    m_in = jnp.concatenate(
        [xd[:, POS:], xs[:, POS:], ea[...], rel_dist,
         jnp.zeros((BE, 1), jnp.float32)], axis=1)
    h1 = jax.nn.silu(mm_r(w1[...], m_in) + b1[...])
    mij = jax.nn.silu(mm_f(w2[...], h1) + b2[...])
    ch = jax.nn.silu(mm_f(cw1[...], mij) + cb1[...])
    cwij = mm_f(cw2[...], ch) + cb2[...]
    gate = jax.nn.sigmoid(mm_f(sw[...], mij) + sb[...])
    m_out = mij * gate
    inv = jax.lax.rsqrt(jnp.maximum(rel_dist, 1e-16))
    mvec = rel * inv * cwij.T * cscale[...]
    p_out[...] = jnp.concatenate(
        [mvec, m_out.T, jnp.zeros((BE, 1), jnp.float32)], axis=1)


def _edge_tc(gs, gd, ea, ew):
    spec_full = lambda a: pl.BlockSpec(a.shape, lambda i: (0,) * a.ndim)
    return pl.pallas_call(
        _edge_body,
        grid=(E // BE,),
        in_specs=[pl.BlockSpec((BE, XD), lambda i: (i, 0)),
                  pl.BlockSpec((BE, XD), lambda i: (i, 0)),
                  pl.BlockSpec((BE, 4), lambda i: (i, 0))]
                 + [spec_full(a) for a in ew],
        out_specs=pl.BlockSpec((BE, PD), lambda i: (i, 0)),
        out_shape=jax.ShapeDtypeStruct((E, PD), jnp.float32),
    )(gs, gd, ea, *ew)


def _node_body(last, x, acc, batch, lnw, lnb, nw1, nb1, nw2,
               gnw, gnb, gnm, fw1, fb1, fw2, fb2, out):
    eps = 1e-5
    xv = x[...]
    a = acc[0] + acc[1]
    coors = xv[:, :POS] + a[:, :POS]
    feats = xv[:, POS:]
    m_i = a[:, POS:POS + MD]

    b2d = batch[...]
    oh = (b2d == lax.broadcasted_iota(jnp.int32, (N, G), 1)
          ).astype(jnp.float32)

    def seg(v):
        return lax.dot_general(oh, v, (((0,), (0,)), ((), ())),
                               preferred_element_type=jnp.float32)

    cnt = jnp.sum(oh, axis=0)
    norm = jnp.maximum(cnt, 1.0) * F
    mean_g = jnp.sum(seg(feats), axis=1) / norm
    xc = feats - jnp.dot(oh, mean_g[:, None],
                         preferred_element_type=jnp.float32)
    var_g = jnp.sum(seg(xc * xc), axis=1) / norm
    rs = jax.lax.rsqrt(var_g + eps)
    fn = xc * jnp.dot(oh, rs[:, None],
                      preferred_element_type=jnp.float32)
    fn = fn * lnw[...] + lnb[...]

    nin = jnp.concatenate([fn, m_i, jnp.zeros((N, 3), jnp.float32)], axis=1)
    h2 = jax.nn.silu(jnp.dot(nin, nw1[...],
                             preferred_element_type=jnp.float32) + nb1[...])
    hid = feats + jnp.dot(h2, nw2[...],
                          preferred_element_type=jnp.float32)[:, :F]
    xg = jnp.concatenate([coors, hid], axis=1)

    mean = jnp.mean(xg, axis=0, keepdims=True)
    og = xg - mean * gnm[...]
    varg = jnp.mean(og * og, axis=0, keepdims=True)
    xn = gnw[...] * og * jax.lax.rsqrt(varg + eps) + gnb[...]

    if last:
        cl = jnp.maximum(cnt, 1.0)
        pooled = seg(xn) / cl[:, None]
        h = jax.nn.relu(jnp.dot(pooled, fw1[...],
                                preferred_element_type=jnp.float32) + fb1[...])
        out[...] = jnp.dot(h, fw2[...],
                           preferred_element_type=jnp.float32) + fb2[...]
    else:
        out[...] = jax.nn.relu(xn)


def _node_tc(x, acc, batch2d, nw, last):
    out_shape = (jax.ShapeDtypeStruct((G, 10), jnp.float32) if last
                 else jax.ShapeDtypeStruct((N, XD), jnp.float32))
    return pl.pallas_call(
        functools.partial(_node_body, last),
        out_shape=out_shape,
    )(x, acc, batch2d, *nw)


def _prep(params):
    layers = []
    for i in range(3):
        p = params["layers"][i]
        gn = params["gn"][i]
        w1 = jnp.zeros((32, 16), jnp.float32).at[:30, :15].set(p["edge_w1"])
        b1 = jnp.zeros((32, 1), jnp.float32).at[:30, 0].set(p["edge_b1"])
        w2 = jnp.zeros((16, 32), jnp.float32).at[:, :30].set(p["edge_w2"])
        b2 = p["edge_b2"][:, None]
        cw1 = p["coors_w1"]
        cb1 = p["coors_b1"][:, None]
        cw2 = p["coors_w2"]
        cb2 = p["coors_b2"][:, None]
        sw = p["soft_w"]
        sb = p["soft_b"][:, None]
        cscale = p["coors_scale"][None, None]
        ew = (w1, b1, w2, b2, cw1, cb1, cw2, cb2, sw, sb, cscale)

        nw1 = jnp.zeros((24, 16), jnp.float32).at[:21, :10].set(p["node_w1"].T)
        nb1 = jnp.zeros((1, 16), jnp.float32).at[0, :10].set(p["node_b1"])
        nw2 = jnp.zeros((16, 8), jnp.float32).at[:10, :5].set(p["node_w2"].T)
        nw = (p["ln_w"][None, :], p["ln_b"][None, :], nw1, nb1, nw2,
              gn["weight"][None, :], gn["bias"][None, :],
              gn["mean_scale"][None, :])
        layers.append((ew, nw))
    (fw1, fb1), (fw2, fb2) = params["fc"]
    head = (fw1.T, fb1[None, :], fw2.T, fb2[None, :])
    return layers, head


def kernel(x, edge_index, batch, edge_attr, params):
    si2 = edge_index[0].reshape(-1, C)
    di2 = edge_index[1].reshape(-1, C)
    z = jnp.zeros((NPT, PD), jnp.float32)
    batch2d = batch[:, None]
    layers, head = _prep(params)

    xcur = x
    res = None
    for i in range(3):
        ew, nw = layers[i]
        gs, gd = _gather(xcur, si2, di2)
        p = _edge_tc(gs, gd, edge_attr, ew)
        acc = _scatter(p, di2, z)
        last = i == 2
        fc = head if last else (jnp.zeros((XD, 32), jnp.float32),
                                jnp.zeros((1, 32), jnp.float32),
                                jnp.zeros((32, 10), jnp.float32),
                                jnp.zeros((1, 10), jnp.float32))
        res = _node_tc(xcur, acc, batch2d, nw + fc, last)
        xcur = res
    return res


# speedup vs baseline: 1.9395x; 1.9395x over previous
import functools

import jax
import jax.numpy as jnp
from jax import lax
from jax.experimental import pallas as pl
from jax.experimental.pallas import tpu as pltpu
from jax.experimental.pallas import tpu_sc as plsc

N = 50000
E = 1600000
G = 128
POS = 3
F = 5
XD = POS + F
MD = 16
PD = 19
NC = 2
NS = 16
NW = NC * NS
C = 128
K = 8
SCH = C * K
EP = 1605632
EPW = EP // NW
NSUP = EPW // SCH
RPW = EPW // C
NPAD = 53248
NPT = NPAD // NS
CHS = 256
NCH = NPT // CHS
DPAD = 50048


def _gather_body(x_hbm, si_hbm, di_hbm, gs_hbm, gd_hbm,
                 xsp, idx_s, idx_d, rows, sem, semw):
    c = lax.axis_index("c")
    s = lax.axis_index("s")
    wid = s * NC + c
    H = SCH // 2

    for t in range(NCH):
        o = s * NPT + t * CHS
        pltpu.sync_copy(x_hbm.at[pl.ds(o, CHS)], rows.at[pl.ds(0, CHS)])
        pltpu.async_copy(rows.at[pl.ds(0, CHS)], xsp.at[pl.ds(o, CHS)],
                         semw).wait()
    pltpu.async_copy(xsp.at[pl.ds(s * NPT, CHS)], rows.at[pl.ds(0, CHS)],
                     sem).wait()
    pltpu.sync_copy(rows.at[pl.ds(0, CHS)], gs_hbm.at[pl.ds(s * NPT, CHS)])

    def step(j, carry):
        r0 = wid * RPW + j * K
        e0 = r0 * C
        pltpu.sync_copy(si_hbm.at[pl.ds(r0, K)], idx_s)
        pltpu.sync_copy(di_hbm.at[pl.ds(r0, K)], idx_d)
        for idx, out in ((idx_s, gs_hbm), (idx_d, gd_hbm)):
            for h in range(2):
                cps = [pltpu.async_copy(xsp.at[idx.at[4 * h + k]],
                                        rows.at[pl.ds(k * C, C)], sem)
                       for k in range(4)]
                for cp in cps:
                    cp.wait()
                pltpu.async_copy(rows, out.at[pl.ds(e0 + h * H, H)],
                                 semw).wait()
        return carry


@functools.cache
def _gather():
    return pl.kernel(
        _gather_body,
        out_type=(jax.ShapeDtypeStruct((EP, XD), jnp.float32),
                  jax.ShapeDtypeStruct((EP, XD), jnp.float32)),
        mesh=plsc.VectorSubcoreMesh(core_axis_name="c", subcore_axis_name="s"),
        scratch_types=[
            pltpu.VMEM_SHARED((NPAD, XD), jnp.float32),
            pltpu.VMEM((K, C), jnp.int32),
            pltpu.VMEM((K, C), jnp.int32),
            pltpu.VMEM((SCH // 2, XD), jnp.float32),
            pltpu.SemaphoreType.DMA,
            pltpu.SemaphoreType.DMA,
        ],
    )


def _scatter_body(p_hbm, di_hbm, z_hbm, out_hbm, idx_d, prow, acc, sem):
    c = lax.axis_index("c")
    s = lax.axis_index("s")
    wid = s * NC + c

    pltpu.sync_copy(z_hbm, prow.at[pl.ds(0, CHS)])
    for t in range(NCH):
        pltpu.sync_copy(prow.at[pl.ds(0, CHS)],
                        acc.at[pl.ds(s * NPT + t * CHS, CHS)])
    plsc.subcore_barrier()

    H = SCH // 2

    def step(j, carry):
        r0 = wid * RPW + j * K
        e0 = r0 * C
        pltpu.sync_copy(di_hbm.at[pl.ds(r0, K)], idx_d)
        for h in range(2):
            pltpu.sync_copy(p_hbm.at[pl.ds(e0 + h * H, H)], prow)
            cps = [pltpu.async_copy(prow.at[pl.ds(k * C, C)],
                                    acc.at[idx_d.at[4 * h + k]], sem,
                                    add=True)
                   for k in range(4)]
            for cp in cps:
                cp.wait()
        return carry

    lax.fori_loop(0, NSUP, step, 0)
    plsc.subcore_barrier()
    for t in range(NCH):
        o = s * NPT + t * CHS
        pltpu.sync_copy(acc.at[pl.ds(o, CHS)], prow.at[pl.ds(0, CHS)])
        pltpu.sync_copy(prow.at[pl.ds(0, CHS)],
                        out_hbm.at[c].at[pl.ds(o, CHS)])


@functools.cache
def _scatter():
    return pl.kernel(
        _scatter_body,
        out_type=jax.ShapeDtypeStruct((NC, NPAD, PD), jnp.float32),
        mesh=plsc.VectorSubcoreMesh(core_axis_name="c", subcore_axis_name="s"),
        scratch_types=[
            pltpu.VMEM((K, C), jnp.int32),
            pltpu.VMEM((SCH // 2, PD), jnp.float32),
            pltpu.VMEM_SHARED((NPAD, PD), jnp.float32),
            pltpu.SemaphoreType.DMA,
        ],
    )


BE = 2000


def _edge_body(gs, gd, ea, w1, b1, w2, b2, cw1, cb1, cw2, cb2, sw, sb,
               cscale, p_out):
    def mm_r(w, x):
        return lax.dot_general(w, x, (((1,), (1,)), ((), ())),
                               preferred_element_type=jnp.float32)

    def mm_f(w, x):
        return lax.dot_general(w, x, (((1,), (0,)), ((), ())),
                               preferred_element_type=jnp.float32)

    xs = gs[...]
    xd = gd[...]
    rel = xs[:, :POS] - xd[:, :POS]
    rel_dist = jnp.sum(rel * rel, axis=1, keepdims=True)
    m_in = jnp.concatenate(
        [xd[:, POS:], xs[:, POS:], ea[...], rel_dist,
         jnp.zeros((BE, 1), jnp.float32)], axis=1)
    h1 = jax.nn.silu(mm_r(w1[...], m_in) + b1[...])
    mij = jax.nn.silu(mm_f(w2[...], h1) + b2[...])
    ch = jax.nn.silu(mm_f(cw1[...], mij) + cb1[...])
    cwij = mm_f(cw2[...], ch) + cb2[...]
    gate = jax.nn.sigmoid(mm_f(sw[...], mij) + sb[...])
    m_out = mij * gate
    inv = jax.lax.rsqrt(jnp.maximum(rel_dist, 1e-16))
    mvec = rel * inv * cwij.T * cscale[...]
    p_out[...] = jnp.concatenate([mvec, m_out.T], axis=1)


def _edge_tc(gs, gd, ea, ew):
    spec_full = lambda a: pl.BlockSpec(a.shape, lambda i: (0,) * a.ndim)
    return pl.pallas_call(
        _edge_body,
        grid=(E // BE,),
        in_specs=[pl.BlockSpec((BE, XD), lambda i: (i, 0)),
                  pl.BlockSpec((BE, XD), lambda i: (i, 0)),
                  pl.BlockSpec((BE, 4), lambda i: (i, 0))]
                 + [spec_full(a) for a in ew],
        out_specs=pl.BlockSpec((BE, PD), lambda i: (i, 0)),
        out_shape=jax.ShapeDtypeStruct((E, PD), jnp.float32),
    )(gs, gd, ea, *ew)


BN = 2000
NB = N // BN
EPS = 1e-5


def _dot(a, b):
    return jnp.dot(a, b, preferred_element_type=jnp.float32)


def _seg(oh, v):
    return lax.dot_general(oh, v, (((0,), (0,)), ((), ())),
                           preferred_element_type=jnp.float32)


def _nstat_body(x, acc, batch, lnw, lnb, nw1, nb1, nw2,
                xg_out, st_out, st0, st1, st2):
    p = pl.program_id(0)
    i = pl.program_id(1)
    oh = (batch[...] == lax.broadcasted_iota(jnp.int32, (BN, G), 1)
          ).astype(jnp.float32)
    xv = x[...]
    feats = xv[:, POS:]

    @pl.when((p == 0) & (i == 0))
    def _():
        st0[...] = jnp.zeros((G, 8), jnp.float32)

    @pl.when(p == 0)
    def _():
        t1 = jnp.sum(feats, axis=1, keepdims=True)
        t2 = jnp.sum(feats * feats, axis=1, keepdims=True)
        z = jnp.concatenate(
            [jnp.ones((BN, 1), jnp.float32), t1, t2,
             jnp.zeros((BN, 5), jnp.float32)], axis=1)
        st0[...] += _seg(oh, z)

    @pl.when((p == 1) & (i == 0))
    def _():
        st1[...] = jnp.zeros((G, 8), jnp.float32)
        st2[...] = jnp.zeros((8, 8), jnp.float32)

    @pl.when(p == 1)
    def _():
        s0 = st0[...]
        cnt = s0[:, :1]
        sa = s0[:, 1:2]
        sb = s0[:, 2:3]
        normc = jnp.maximum(cnt, 1.0) * F
        m = sa / normc
        var_g = (sb - 2.0 * m * sa + F * cnt * m * m) / normc
        rsg = jax.lax.rsqrt(var_g + EPS)
        m_pn = _dot(oh, m)
        rs_pn = _dot(oh, rsg)
        fn = (feats - m_pn) * rs_pn * lnw[...] + lnb[...]
        a = acc[...]
        nin = jnp.concatenate(
            [fn, a[:, POS:POS + MD], jnp.zeros((BN, 3), jnp.float32)], axis=1)
        h2 = jax.nn.silu(_dot(nin, nw1[...]) + nb1[...])
        hid = feats + _dot(h2, nw2[...])[:, :F]
        xg = jnp.concatenate([xv[:, :POS] + a[:, :POS], hid], axis=1)
        xg_out[...] = xg
        st1[...] += _seg(oh, xg)
        st2[0:1, :] += jnp.sum(xg, axis=0, keepdims=True)
        st2[1:2, :] += jnp.sum(xg * xg, axis=0, keepdims=True)

    @pl.when((p == 1) & (i == NB - 1))
    def _():
        st_out[...] = jnp.concatenate([st1[...], st0[...], st2[...]], axis=0)


def _nstat_tc(x, acc, batch2d, w):
    full = lambda a: pl.BlockSpec(a.shape, lambda p, i: (0,) * a.ndim)
    return pl.pallas_call(
        _nstat_body,
        grid=(2, NB),
        in_specs=[pl.BlockSpec((BN, XD), lambda p, i: (i, 0)),
                  pl.BlockSpec((BN, PD), lambda p, i: (i, 0)),
                  pl.BlockSpec((BN, 1), lambda p, i: (i, 0))]
                 + [full(a) for a in w],
        out_specs=[pl.BlockSpec((BN, XD), lambda p, i: (i, 0)),
                   pl.BlockSpec((G + G + 8, XD), lambda p, i: (0, 0))],
        out_shape=[jax.ShapeDtypeStruct((N, XD), jnp.float32),
                   jax.ShapeDtypeStruct((G + G + 8, XD), jnp.float32)],
        scratch_shapes=[pltpu.VMEM((G, 8), jnp.float32),
                        pltpu.VMEM((G, 8), jnp.float32),
                        pltpu.VMEM((8, 8), jnp.float32)],
    )(x, acc, batch2d, *w)


def _gn_scale(st, gnm, gnw):
    s1 = st[G + G:G + G + 1, :]
    s2 = st[G + G + 1:G + G + 2, :]
    mu = (s1 / N) * gnm
    varg = s2 / N - 2.0 * mu * s1 / N + mu * mu
    return mu, gnw * jax.lax.rsqrt(varg + EPS)


def _gnapply_body(xg, st, gnw, gnb, gnm, out):
    mu, sg = _gn_scale(st[...], gnm[...], gnw[...])
    out[...] = jax.nn.relu(sg * (xg[...] - mu) + gnb[...])


def _gnapply_tc(xg, st, gn):
    full = lambda a: pl.BlockSpec(a.shape, lambda i: (0,) * a.ndim)
    return pl.pallas_call(
        _gnapply_body,
        grid=(NB,),
        in_specs=[pl.BlockSpec((BN, XD), lambda i: (i, 0))]
                 + [full(a) for a in (st,) + gn],
        out_specs=pl.BlockSpec((BN, XD), lambda i: (i, 0)),
        out_shape=jax.ShapeDtypeStruct((N, XD), jnp.float32),
    )(xg, st, *gn)


def _head_body(st, gnw, gnb, gnm, fw1, fb1, fw2, fb2, out):
    s = st[...]
    mu, sg = _gn_scale(s, gnm[...], gnw[...])
    seg_xg = s[:G, :]
    cnt = s[G:G + G, :1]
    seg_xn = sg * (seg_xg - mu * cnt) + gnb[...] * cnt
    pooled = seg_xn / jnp.maximum(cnt, 1.0)
    h = jax.nn.relu(_dot(pooled, fw1[...]) + fb1[...])
    out[...] = _dot(h, fw2[...]) + fb2[...]


def _head_tc(st, hw):
    return pl.pallas_call(
        _head_body,
        out_shape=jax.ShapeDtypeStruct((G, 10), jnp.float32),
    )(st, *hw)


def _prep(params):
    layers = []
    for i in range(3):
        p = params["layers"][i]
        gn = params["gn"][i]
        w1 = jnp.zeros((32, 16), jnp.float32).at[:30, :15].set(p["edge_w1"])
        b1 = jnp.zeros((32, 1), jnp.float32).at[:30, 0].set(p["edge_b1"])
        w2 = jnp.zeros((16, 32), jnp.float32).at[:, :30].set(p["edge_w2"])
        b2 = p["edge_b2"][:, None]
        cw1 = p["coors_w1"]
        cb1 = p["coors_b1"][:, None]
        cw2 = p["coors_w2"]
        cb2 = p["coors_b2"][:, None]
        sw = p["soft_w"]
        sb = p["soft_b"][:, None]
        cscale = p["coors_scale"][None, None]
        ew = (w1, b1, w2, b2, cw1, cb1, cw2, cb2, sw, sb, cscale)

        nw1 = jnp.zeros((24, 16), jnp.float32).at[:21, :10].set(p["node_w1"].T)
        nb1 = jnp.zeros((1, 16), jnp.float32).at[0, :10].set(p["node_b1"])
        nw2 = jnp.zeros((16, 8), jnp.float32).at[:10, :5].set(p["node_w2"].T)
        nw = (p["ln_w"][None, :], p["ln_b"][None, :], nw1, nb1, nw2)
        gnp = (gn["weight"][None, :], gn["bias"][None, :],
               gn["mean_scale"][None, :])
        layers.append((ew, nw, gnp))
    (fw1, fb1), (fw2, fb2) = params["fc"]
    head = (fw1.T, fb1[None, :], fw2.T, fb2[None, :])
    return layers, head


def kernel(x, edge_index, batch, edge_attr, params):
    src = edge_index[0]
    dst = edge_index[1]
    batch2d = batch[:, None]
    layers, head = _prep(params)

    xcur = x
    res = None
    for i in range(3):
        ew, nw, gnp = layers[i]
        gs = xcur[src]
        gd = xcur[dst]
        p = _edge_tc(gs, gd, edge_attr, ew)
        acc = jax.ops.segment_sum(p, dst, num_segments=N)
        xg, st = _nstat_tc(xcur, acc, batch2d, nw)
        if i == 2:
            res = _head_tc(st, gnp + head)
        else:
            res = _gnapply_tc(xg, st, gnp)
        xcur = res
    return res
